# baseline (speedup 1.0000x reference)
import functools

import jax
import jax.numpy as jnp
from jax import lax
from jax.experimental import pallas as pl
from jax.experimental.pallas import tpu as pltpu
from jax.experimental.pallas import tpu_sc as plsc

VOCAB = 1000000
EMB = 64
B = 4096
L = 200
NTOK = B * L
NC = 2
NS = 16
NW = NC * NS

FULLBLK = 7812
TAILW = VOCAB - FULLBLK * 128
NGRP = FULLBLK // 4
GMAX = 62

ROWS_PER_W = NTOK // NW
CHUNK = 128
NCHUNK = ROWS_PER_W // CHUNK
BPL = B // (4 * CHUNK)

_mesh = plsc.VectorSubcoreMesh(core_axis_name="c", subcore_axis_name="s")
_params = pltpu.CompilerParams(
    use_tc_tiling_on_sc=True, needs_layout_passes=False
)


def _wid():
    return lax.axis_index("s") * NC + lax.axis_index("c")


@functools.partial(
    pl.kernel,
    mesh=_mesh,
    out_type=jax.ShapeDtypeStruct((VOCAB, 128), jnp.float32),
    scratch_types=[
        pltpu.VMEM((EMB, 512), jnp.float32),
        pltpu.VMEM((EMB, 512), jnp.float32),
        pltpu.VMEM((128, 128), jnp.float32),
        pltpu.VMEM((128, 128), jnp.float32),
        pltpu.VMEM((TAILW * EMB,), jnp.float32),
        pltpu.SemaphoreType.DMA,
        pltpu.SemaphoreType.DMA,
        pltpu.SemaphoreType.DMA,
        pltpu.SemaphoreType.DMA,
    ],
    compiler_params=_params,
)
def _transpose_sc(tt_hbm, tail_hbm, scr_hbm, sv0, sv1, dv0, dv1,
                  tl_v, ss0, ss1, ds0, ds1):
    wid = _wid()
    srcs = (sv0, sv1)
    dsts = (dv0, dv1)
    ssems = (ss0, ss1)
    dsems = (ds0, ds1)
    base16 = lax.iota(jnp.int32, 16)

    def load_group(g, s):
        off = pl.multiple_of(g * 512, 512)
        for fr in range(8):
            pltpu.async_copy(
                tt_hbm.at[pl.ds(fr * 8, 8), pl.ds(off, 512)],
                srcs[s].at[pl.ds(fr * 8, 8)],
                ssems[s],
            )

    def wait_load(s):
        for fr in range(8):
            pltpu.make_async_copy(
                tt_hbm.at[pl.ds(0, 8), pl.ds(0, 512)],
                srcs[s].at[pl.ds(fr * 8, 8)],
                ssems[s],
            ).wait()

    def store_block(c, sd):
        off = pl.multiple_of(c * 128, 128)
        pltpu.async_copy(dsts[sd], scr_hbm.at[pl.ds(off, 128)], dsems[sd])

    def wait_store(sd):
        pltpu.make_async_copy(
            dsts[sd], scr_hbm.at[pl.ds(0, 128)], dsems[sd]
        ).wait()

    def transpose_block(s, cc, sd):
        @plsc.parallel_loop(0, 128, unroll=8)
        def row(j):
            jv = base16 * 0 + (cc * 128 + j)
            for i4 in range(4):
                vals = plsc.load_gather(srcs[s], [base16 + i4 * 16, jv])
                dsts[sd][j, pl.ds(i4 * 16, 16)] = vals

    load_group(wid, 0)

    def body(r, carry):
        for s in range(2):
            m = r * 2 + s
            g = wid + m * NW
            g2 = wid + (m + 1) * NW

            @pl.when(g2 < NGRP)
            def _():
                load_group(g2, 1 - s)

            @pl.when(g < NGRP)
            def _():
                wait_load(s)
                for cc in range(4):
                    sd = cc % 2

                    @pl.when(m * 4 + cc >= 2)
                    def _():
                        wait_store(sd)

                    transpose_block(s, cc, sd)
                    store_block(g * 4 + cc, sd)

        return carry

    lax.fori_loop(0, GMAX // 2, body, 0)
    wait_store(0)
    wait_store(1)

    @pl.when(wid == NGRP % NW)
    def _():
        pltpu.sync_copy(tail_hbm, tl_v)

        def rowt(j, carry):
            for x0 in (0, 16, 32, 48):
                vals = plsc.load_gather(tl_v, [j * EMB + base16 + x0])
                dv0[j, pl.ds(x0, 16)] = vals
            return carry

        lax.fori_loop(0, TAILW, rowt, 0)
        pltpu.sync_copy(
            dv0.at[pl.ds(0, TAILW)], scr_hbm.at[pl.ds(FULLBLK * 128, TAILW)]
        )


@functools.partial(
    pl.kernel,
    mesh=_mesh,
    out_type=jax.ShapeDtypeStruct((L, EMB, B), jnp.float32),
    scratch_types=[
        pltpu.VMEM((ROWS_PER_W,), jnp.int32),
        pltpu.VMEM((CHUNK, 128), jnp.float32),
        pltpu.VMEM((CHUNK, 128), jnp.float32),
        pltpu.VMEM((1, EMB, 4 * CHUNK), jnp.float32),
        pltpu.VMEM((1, EMB, 4 * CHUNK), jnp.float32),
        pltpu.SemaphoreType.DMA,
        pltpu.SemaphoreType.DMA,
        pltpu.SemaphoreType.DMA,
        pltpu.SemaphoreType.DMA,
    ],
    compiler_params=_params,
)
def _gather_sc(tokens_hbm, scr_hbm, out_hbm, idx_all, rv0, rv1, tb0, tb1,
               gs0, gs1, os0, os1):
    wid = _wid()
    cbase = wid * NCHUNK
    rows = (rv0, rv1)
    tbufs = (tb0, tb1)
    gsems = (gs0, gs1)
    osems = (os0, os1)
    base16 = lax.iota(jnp.int32, 16)
    groups = [base16 + g * 16 for g in range(CHUNK // 16)]

    pltpu.sync_copy(
        tokens_hbm.at[pl.ds(pl.multiple_of(cbase * CHUNK, CHUNK), ROWS_PER_W)],
        idx_all,
    )

    def gather(j, s):
        pltpu.async_copy(
            scr_hbm.at[idx_all.at[pl.ds(j * CHUNK, CHUNK)]], rows[s], gsems[s]
        )

    def wait_gather(s):
        pltpu.make_async_copy(
            scr_hbm.at[pl.ds(0, CHUNK)], rows[s], gsems[s]
        ).wait()

    def transpose_chunk(s, u, st):
        @plsc.parallel_loop(0, EMB, unroll=4)
        def frow(f):
            fv = base16 * 0 + f
            for g in range(CHUNK // 16):
                vals = plsc.load_gather(rows[s], [groups[g], fv])
                tbufs[st][0, f, pl.ds(u * CHUNK + g * 16, 16)] = vals

    def put(r, st):
        grp = cbase // 4 + r
        l = grp // BPL
        b0 = pl.multiple_of((grp % BPL) * 4 * CHUNK, 4 * CHUNK)
        pltpu.async_copy(
            tbufs[st],
            out_hbm.at[pl.ds(l, 1), :, pl.ds(b0, 4 * CHUNK)],
            osems[st],
        )

    def wait_put(st):
        pltpu.make_async_copy(
            tbufs[st],
            out_hbm.at[pl.ds(0, 1), :, pl.ds(0, 4 * CHUNK)],
            osems[st],
        ).wait()

    gather(0, 0)
    gather(1, 1)

    def body(r2, carry):
        for st in range(2):
            r = r2 * 2 + st

            @pl.when(r >= 2)
            def _():
                wait_put(st)

            for u in range(4):
                j = r * 4 + u
                s = u % 2
                wait_gather(s)
                transpose_chunk(s, u, st)

                @pl.when(j + 2 < NCHUNK)
                def _():
                    gather(j + 2, s)

            put(r, st)
        return carry

    lax.fori_loop(0, NCHUNK // 8, body, 0)
    wait_put(0)
    wait_put(1)


def kernel(tokens, table):
    tt = jnp.swapaxes(table, 0, 1)
    tail = table[FULLBLK * 128:].reshape(TAILW * EMB)
    scratch = _transpose_sc(tt, tail)
    flat = jnp.swapaxes(tokens, 0, 1).reshape(NTOK).astype(jnp.int32)
    out = _gather_sc(flat, scratch)
    return jnp.transpose(out, (2, 0, 1))


# speedup vs baseline: 1.4146x; 1.4146x over previous
import functools

import jax
import jax.numpy as jnp
from jax import lax
from jax.experimental import pallas as pl
from jax.experimental.pallas import tpu as pltpu
from jax.experimental.pallas import tpu_sc as plsc

VOCAB = 1000000
EMB = 64
B = 4096
L = 200
NTOK = B * L
NC = 2
NS = 16
NW = NC * NS

FULLBLK = 7812
TAILW = VOCAB - FULLBLK * 128
NGRP = FULLBLK // 4
GMAX = 62

ROWS_PER_W = NTOK // NW
CHUNK = 128
NCHUNK = ROWS_PER_W // CHUNK
BPL = B // (4 * CHUNK)

_mesh = plsc.VectorSubcoreMesh(core_axis_name="c", subcore_axis_name="s")
_params = pltpu.CompilerParams(
    use_tc_tiling_on_sc=True, needs_layout_passes=False
)


def _wid():
    return lax.axis_index("s") * NC + lax.axis_index("c")


@functools.partial(
    pl.kernel,
    mesh=_mesh,
    out_type=jax.ShapeDtypeStruct((VOCAB, 128), jnp.float32),
    scratch_types=[
        pltpu.VMEM((EMB, 128), jnp.float32),
        pltpu.VMEM((EMB, 128), jnp.float32),
        pltpu.VMEM((128, 128), jnp.float32),
        pltpu.VMEM((128, 128), jnp.float32),
        pltpu.VMEM((TAILW * EMB,), jnp.float32),
        pltpu.SemaphoreType.DMA,
        pltpu.SemaphoreType.DMA,
        pltpu.SemaphoreType.DMA,
        pltpu.SemaphoreType.DMA,
    ],
    compiler_params=_params,
)
def _transpose_sc(tt_hbm, tail_hbm, scr_hbm, sv0, sv1, dv0, dv1,
                  tl_v, ss0, ss1, ds0, ds1):
    wid = _wid()
    srcs = (sv0, sv1)
    dsts = (dv0, dv1)
    ssems = (ss0, ss1)
    dsems = (ds0, ds1)
    base16 = lax.iota(jnp.int32, 16)

    def load_block(c, s):
        off = pl.multiple_of(c * 128, 128)
        pltpu.async_copy(tt_hbm.at[:, pl.ds(off, 128)], srcs[s], ssems[s])

    def wait_load(s):
        pltpu.make_async_copy(
            tt_hbm.at[:, pl.ds(0, 128)], srcs[s], ssems[s]
        ).wait()

    def store_block(c, s):
        off = pl.multiple_of(c * 128, 128)
        pltpu.async_copy(dsts[s], scr_hbm.at[pl.ds(off, 128)], dsems[s])

    def wait_store(s):
        pltpu.make_async_copy(
            dsts[s], scr_hbm.at[pl.ds(0, 128)], dsems[s]
        ).wait()

    perm = [(base16 + d) & 15 for d in range(16)]
    fvecs = [base16 + f0 for f0 in (0, 16, 32, 48)]

    def transpose_block(s):
        @plsc.parallel_loop(0, 128, step=16)
        def row(j0):
            for i4 in range(4):
                for d in range(16):
                    jv = perm[d] + j0
                    vals = plsc.load_gather(srcs[s], [fvecs[i4], jv])
                    plsc.store_scatter(dsts[s], [jv, fvecs[i4]], vals)

    load_block(wid, 0)

    def body(r, carry):
        for s in range(2):
            k = r * 2 + s
            c = wid + k * NW
            c2 = wid + (k + 1) * NW

            @pl.when(c2 < FULLBLK)
            def _():
                load_block(c2, 1 - s)

            @pl.when(c < FULLBLK)
            def _():
                wait_load(s)

                @pl.when(k >= 2)
                def _():
                    wait_store(s)

                transpose_block(s)
                store_block(c, s)

        return carry

    lax.fori_loop(0, 123, body, 0)
    wait_store(0)
    wait_store(1)

    @pl.when(wid == NGRP % NW)
    def _():
        pltpu.sync_copy(tail_hbm, tl_v)

        def rowt(j, carry):
            for x0 in (0, 16, 32, 48):
                vals = plsc.load_gather(tl_v, [j * EMB + base16 + x0])
                dv0[j, pl.ds(x0, 16)] = vals
            return carry

        lax.fori_loop(0, TAILW, rowt, 0)
        pltpu.sync_copy(
            dv0.at[pl.ds(0, TAILW)], scr_hbm.at[pl.ds(FULLBLK * 128, TAILW)]
        )


@functools.partial(
    pl.kernel,
    mesh=_mesh,
    out_type=jax.ShapeDtypeStruct((L, EMB, B), jnp.float32),
    scratch_types=[
        pltpu.VMEM((ROWS_PER_W,), jnp.int32),
        pltpu.VMEM((CHUNK, 128), jnp.float32),
        pltpu.VMEM((CHUNK, 128), jnp.float32),
        pltpu.VMEM((1, EMB, CHUNK), jnp.float32),
        pltpu.VMEM((1, EMB, CHUNK), jnp.float32),
        pltpu.SemaphoreType.DMA,
        pltpu.SemaphoreType.DMA,
        pltpu.SemaphoreType.DMA,
        pltpu.SemaphoreType.DMA,
    ],
    compiler_params=_params,
)
def _gather_sc(tokens_hbm, scr_hbm, out_hbm, idx_all, rv0, rv1, tb0, tb1,
               gs0, gs1, os0, os1):
    wid = _wid()
    cbase = wid * NCHUNK
    rows = (rv0, rv1)
    tbufs = (tb0, tb1)
    gsems = (gs0, gs1)
    osems = (os0, os1)
    base16 = lax.iota(jnp.int32, 16)
    groups = [base16 + g * 16 for g in range(CHUNK // 16)]

    pltpu.sync_copy(
        tokens_hbm.at[pl.ds(pl.multiple_of(cbase * CHUNK, CHUNK), ROWS_PER_W)],
        idx_all,
    )

    def gather(j, s):
        pltpu.async_copy(
            scr_hbm.at[idx_all.at[pl.ds(j * CHUNK, CHUNK)]], rows[s], gsems[s]
        )

    def wait_gather(s):
        pltpu.make_async_copy(
            scr_hbm.at[pl.ds(0, CHUNK)], rows[s], gsems[s]
        ).wait()

    perm = [(base16 + d) & 15 for d in range(16)]
    fvecs = [base16 + f0 for f0 in (0, 16, 32, 48)]
    zero16 = base16 * 0

    def transpose_chunk(s, st):
        @plsc.parallel_loop(0, CHUNK, step=16)
        def blk(j0):
            jv = base16 + j0
            for i4 in range(4):
                for d in range(16):
                    fv = perm[d] + i4 * 16
                    vals = plsc.load_gather(rows[s], [jv, fv])
                    plsc.store_scatter(tbufs[st], [zero16, fv, jv], vals)

    def put(j, st):
        ch = cbase + j
        l = ch // (BPL * 4)
        b0 = pl.multiple_of((ch % (BPL * 4)) * CHUNK, CHUNK)
        pltpu.async_copy(
            tbufs[st],
            out_hbm.at[pl.ds(l, 1), :, pl.ds(b0, CHUNK)],
            osems[st],
        )

    def wait_put(st):
        pltpu.make_async_copy(
            tbufs[st],
            out_hbm.at[pl.ds(0, 1), :, pl.ds(0, CHUNK)],
            osems[st],
        ).wait()

    gather(0, 0)
    gather(1, 1)

    def body(r, carry):
        for s in range(2):
            j = r * 2 + s
            wait_gather(s)

            @pl.when(j >= 2)
            def _():
                wait_put(s)

            transpose_chunk(s, s)

            @pl.when(j + 2 < NCHUNK)
            def _():
                gather(j + 2, s)

            put(j, s)
        return carry

    lax.fori_loop(0, NCHUNK // 2, body, 0)
    wait_put(0)
    wait_put(1)


def kernel(tokens, table):
    tt = jnp.swapaxes(table, 0, 1)
    tail = table[FULLBLK * 128:].reshape(TAILW * EMB)
    scratch = _transpose_sc(tt, tail)
    flat = jnp.swapaxes(tokens, 0, 1).reshape(NTOK).astype(jnp.int32)
    out = _gather_sc(flat, scratch)
    return jnp.transpose(out, (2, 0, 1))


# speedup vs baseline: 1.7945x; 1.2685x over previous
import functools

import jax
import jax.numpy as jnp
from jax import lax
from jax.experimental import pallas as pl
from jax.experimental.pallas import tpu as pltpu
from jax.experimental.pallas import tpu_sc as plsc

VOCAB = 1000000
EMB = 64
B = 4096
L = 200
NTOK = B * L
NC = 2
NS = 16
NW = NC * NS

FULLBLK = 7812
TAILW = VOCAB - FULLBLK * 128
NGRP = FULLBLK // 4
GMAX = 62

ROWS_PER_W = NTOK // NW
CHUNK = 128
NCHUNK = ROWS_PER_W // CHUNK
BPL = B // (4 * CHUNK)

_mesh = plsc.VectorSubcoreMesh(core_axis_name="c", subcore_axis_name="s")
_params = pltpu.CompilerParams(
    use_tc_tiling_on_sc=True, needs_layout_passes=False
)


def _wid():
    return lax.axis_index("s") * NC + lax.axis_index("c")


@functools.partial(
    pl.kernel,
    mesh=_mesh,
    out_type=jax.ShapeDtypeStruct((VOCAB, 128), jnp.float32),
    scratch_types=[
        pltpu.VMEM((EMB, 128), jnp.float32),
        pltpu.VMEM((EMB, 128), jnp.float32),
        pltpu.VMEM((128, 128), jnp.float32),
        pltpu.VMEM((128, 128), jnp.float32),
        pltpu.VMEM((TAILW * EMB,), jnp.float32),
        pltpu.SemaphoreType.DMA,
        pltpu.SemaphoreType.DMA,
        pltpu.SemaphoreType.DMA,
        pltpu.SemaphoreType.DMA,
    ],
    compiler_params=_params,
)
def _transpose_sc(tt_hbm, tail_hbm, scr_hbm, sv0, sv1, dv0, dv1,
                  tl_v, ss0, ss1, ds0, ds1):
    wid = _wid()
    srcs = (sv0, sv1)
    dsts = (dv0, dv1)
    ssems = (ss0, ss1)
    dsems = (ds0, ds1)
    base16 = lax.iota(jnp.int32, 16)

    def load_block(c, s):
        off = pl.multiple_of(c * 128, 128)
        pltpu.async_copy(tt_hbm.at[:, pl.ds(off, 128)], srcs[s], ssems[s])

    def wait_load(s):
        pltpu.make_async_copy(
            tt_hbm.at[:, pl.ds(0, 128)], srcs[s], ssems[s]
        ).wait()

    def store_block(c, s):
        off = pl.multiple_of(c * 128, 128)
        pltpu.async_copy(dsts[s], scr_hbm.at[pl.ds(off, 128)], dsems[s])

    def wait_store(s):
        pltpu.make_async_copy(
            dsts[s], scr_hbm.at[pl.ds(0, 128)], dsems[s]
        ).wait()

    perm = [(base16 + d) & 15 for d in range(16)]
    fvecs = [base16 + f0 for f0 in (0, 16, 32, 48)]

    def transpose_block(s):
        @plsc.parallel_loop(0, 128, step=16, unroll=2)
        def row(j0):
            for i4 in range(4):
                for d in range(16):
                    jv = perm[d] + j0
                    vals = plsc.load_gather(srcs[s], [fvecs[i4], jv])
                    plsc.store_scatter(dsts[s], [jv, fvecs[i4]], vals)

    load_block(wid, 0)

    def body(r, carry):
        for s in range(2):
            k = r * 2 + s
            c = wid + k * NW
            c2 = wid + (k + 1) * NW

            @pl.when(c2 < FULLBLK)
            def _():
                load_block(c2, 1 - s)

            @pl.when(c < FULLBLK)
            def _():
                wait_load(s)

                @pl.when(k >= 2)
                def _():
                    wait_store(s)

                transpose_block(s)
                store_block(c, s)

        return carry

    lax.fori_loop(0, 123, body, 0)
    wait_store(0)
    wait_store(1)

    @pl.when(wid == NGRP % NW)
    def _():
        pltpu.sync_copy(tail_hbm, tl_v)

        def rowt(j, carry):
            for x0 in (0, 16, 32, 48):
                vals = plsc.load_gather(tl_v, [j * EMB + base16 + x0])
                dv0[j, pl.ds(x0, 16)] = vals
            return carry

        lax.fori_loop(0, TAILW, rowt, 0)
        pltpu.sync_copy(
            dv0.at[pl.ds(0, TAILW)], scr_hbm.at[pl.ds(FULLBLK * 128, TAILW)]
        )


@functools.partial(
    pl.kernel,
    mesh=_mesh,
    out_type=jax.ShapeDtypeStruct((L, EMB, B), jnp.float32),
    scratch_types=[
        pltpu.VMEM((ROWS_PER_W,), jnp.int32),
        pltpu.VMEM((CHUNK, 128), jnp.float32),
        pltpu.VMEM((CHUNK, 128), jnp.float32),
        pltpu.VMEM((1, EMB, CHUNK), jnp.float32),
        pltpu.VMEM((1, EMB, CHUNK), jnp.float32),
        pltpu.SemaphoreType.DMA,
        pltpu.SemaphoreType.DMA,
        pltpu.SemaphoreType.DMA,
        pltpu.SemaphoreType.DMA,
    ],
    compiler_params=_params,
)
def _gather_sc(tokens_hbm, scr_hbm, out_hbm, idx_all, rv0, rv1, tb0, tb1,
               gs0, gs1, os0, os1):
    wid = _wid()
    cbase = wid * NCHUNK
    rows = (rv0, rv1)
    tbufs = (tb0, tb1)
    gsems = (gs0, gs1)
    osems = (os0, os1)
    base16 = lax.iota(jnp.int32, 16)
    groups = [base16 + g * 16 for g in range(CHUNK // 16)]

    pltpu.sync_copy(
        tokens_hbm.at[pl.ds(pl.multiple_of(cbase * CHUNK, CHUNK), ROWS_PER_W)],
        idx_all,
    )

    def gather(j, s):
        pltpu.async_copy(
            scr_hbm.at[idx_all.at[pl.ds(j * CHUNK, CHUNK)]], rows[s], gsems[s]
        )

    def wait_gather(s):
        pltpu.make_async_copy(
            scr_hbm.at[pl.ds(0, CHUNK)], rows[s], gsems[s]
        ).wait()

    perm = [(base16 + d) & 15 for d in range(16)]
    fvecs = [base16 + f0 for f0 in (0, 16, 32, 48)]
    zero16 = base16 * 0

    def transpose_chunk(s, st):
        @plsc.parallel_loop(0, CHUNK, step=16, unroll=2)
        def blk(j0):
            jv = base16 + j0
            for i4 in range(4):
                for d in range(16):
                    fv = perm[d] + i4 * 16
                    vals = plsc.load_gather(rows[s], [jv, fv])
                    plsc.store_scatter(tbufs[st], [zero16, fv, jv], vals)

    def put(j, st):
        ch = cbase + j
        l = ch // (BPL * 4)
        b0 = pl.multiple_of((ch % (BPL * 4)) * CHUNK, CHUNK)
        pltpu.async_copy(
            tbufs[st],
            out_hbm.at[pl.ds(l, 1), :, pl.ds(b0, CHUNK)],
            osems[st],
        )

    def wait_put(st):
        pltpu.make_async_copy(
            tbufs[st],
            out_hbm.at[pl.ds(0, 1), :, pl.ds(0, CHUNK)],
            osems[st],
        ).wait()

    gather(0, 0)
    gather(1, 1)

    def body(r, carry):
        for s in range(2):
            j = r * 2 + s
            wait_gather(s)

            @pl.when(j >= 2)
            def _():
                wait_put(s)

            transpose_chunk(s, s)

            @pl.when(j + 2 < NCHUNK)
            def _():
                gather(j + 2, s)

            put(j, s)
        return carry

    lax.fori_loop(0, NCHUNK // 2, body, 0)
    wait_put(0)
    wait_put(1)


def kernel(tokens, table):
    tt = jnp.swapaxes(table, 0, 1)
    tail = table[FULLBLK * 128:].reshape(TAILW * EMB)
    scratch = _transpose_sc(tt, tail)
    flat = jnp.swapaxes(tokens, 0, 1).reshape(NTOK).astype(jnp.int32)
    out = _gather_sc(flat, scratch)
    return jnp.transpose(out, (2, 0, 1))


# speedup vs baseline: 2.6038x; 1.4510x over previous
import functools

import jax
import jax.numpy as jnp
from jax import lax
from jax.experimental import pallas as pl
from jax.experimental.pallas import tpu as pltpu
from jax.experimental.pallas import tpu_sc as plsc

VOCAB = 1000000
EMB = 64
B = 4096
L = 200
NTOK = B * L
NC = 2
NS = 16
NW = NC * NS

FULLBLK = 7812
TAILW = VOCAB - FULLBLK * 128
NGRP = FULLBLK // 4
GMAX = 62

ROWS_PER_W = NTOK // NW
CHUNK = 128
NCHUNK = ROWS_PER_W // CHUNK
BPL = B // (4 * CHUNK)

_mesh = plsc.VectorSubcoreMesh(core_axis_name="c", subcore_axis_name="s")
_params = pltpu.CompilerParams(
    use_tc_tiling_on_sc=True, needs_layout_passes=False
)


def _wid():
    return lax.axis_index("s") * NC + lax.axis_index("c")


@functools.partial(
    pl.kernel,
    mesh=_mesh,
    out_type=jax.ShapeDtypeStruct((VOCAB, 128), jnp.float32),
    scratch_types=[
        pltpu.VMEM((EMB, 128), jnp.float32),
        pltpu.VMEM((EMB, 128), jnp.float32),
        pltpu.VMEM((128, 128), jnp.float32),
        pltpu.VMEM((128, 128), jnp.float32),
        pltpu.VMEM((TAILW * EMB,), jnp.float32),
        pltpu.SemaphoreType.DMA,
        pltpu.SemaphoreType.DMA,
        pltpu.SemaphoreType.DMA,
        pltpu.SemaphoreType.DMA,
    ],
    compiler_params=_params,
)
def _transpose_sc(tt_hbm, tail_hbm, scr_hbm, sv0, sv1, dv0, dv1,
                  tl_v, ss0, ss1, ds0, ds1):
    wid = _wid()
    srcs = (sv0, sv1)
    dsts = (dv0, dv1)
    ssems = (ss0, ss1)
    dsems = (ds0, ds1)
    base16 = lax.iota(jnp.int32, 16)

    def load_block(c, s):
        off = pl.multiple_of(c * 128, 128)
        pltpu.async_copy(tt_hbm.at[:, pl.ds(off, 128)], srcs[s], ssems[s])

    def wait_load(s):
        pltpu.make_async_copy(
            tt_hbm.at[:, pl.ds(0, 128)], srcs[s], ssems[s]
        ).wait()

    def store_block(c, s):
        off = pl.multiple_of(c * 128, 128)
        pltpu.async_copy(dsts[s], scr_hbm.at[pl.ds(off, 128)], dsems[s])

    def wait_store(s):
        pltpu.make_async_copy(
            dsts[s], scr_hbm.at[pl.ds(0, 128)], dsems[s]
        ).wait()

    perm = [(base16 + d) & 15 for d in range(16)]
    fvecs = [base16 + f0 for f0 in (0, 16, 32, 48)]

    def transpose_block(s):
        @plsc.parallel_loop(0, 128, step=16, unroll=4)
        def row(j0):
            for i4 in range(4):
                for d in range(16):
                    jv = perm[d] + j0
                    vals = plsc.load_gather(srcs[s], [fvecs[i4], jv])
                    plsc.store_scatter(dsts[s], [jv, fvecs[i4]], vals)

    load_block(wid, 0)

    def body(r, carry):
        for s in range(2):
            k = r * 2 + s
            c = wid + k * NW
            c2 = wid + (k + 1) * NW

            @pl.when(c2 < FULLBLK)
            def _():
                load_block(c2, 1 - s)

            @pl.when(c < FULLBLK)
            def _():
                wait_load(s)

                @pl.when(k >= 2)
                def _():
                    wait_store(s)

                transpose_block(s)
                store_block(c, s)

        return carry

    lax.fori_loop(0, 123, body, 0)
    wait_store(0)
    wait_store(1)

    @pl.when(wid == NGRP % NW)
    def _():
        pltpu.sync_copy(tail_hbm, tl_v)

        def rowt(j, carry):
            for x0 in (0, 16, 32, 48):
                vals = plsc.load_gather(tl_v, [j * EMB + base16 + x0])
                dv0[j, pl.ds(x0, 16)] = vals
            return carry

        lax.fori_loop(0, TAILW, rowt, 0)
        pltpu.sync_copy(
            dv0.at[pl.ds(0, TAILW)], scr_hbm.at[pl.ds(FULLBLK * 128, TAILW)]
        )


@functools.partial(
    pl.kernel,
    mesh=_mesh,
    out_type=jax.ShapeDtypeStruct((L, EMB, B), jnp.float32),
    scratch_types=[
        pltpu.VMEM((ROWS_PER_W,), jnp.int32),
        pltpu.VMEM((CHUNK, 128), jnp.float32),
        pltpu.VMEM((CHUNK, 128), jnp.float32),
        pltpu.VMEM((1, EMB, CHUNK), jnp.float32),
        pltpu.VMEM((1, EMB, CHUNK), jnp.float32),
        pltpu.SemaphoreType.DMA,
        pltpu.SemaphoreType.DMA,
        pltpu.SemaphoreType.DMA,
        pltpu.SemaphoreType.DMA,
    ],
    compiler_params=_params,
)
def _gather_sc(tokens_hbm, scr_hbm, out_hbm, idx_all, rv0, rv1, tb0, tb1,
               gs0, gs1, os0, os1):
    wid = _wid()
    cbase = wid * NCHUNK
    rows = (rv0, rv1)
    tbufs = (tb0, tb1)
    gsems = (gs0, gs1)
    osems = (os0, os1)
    base16 = lax.iota(jnp.int32, 16)
    groups = [base16 + g * 16 for g in range(CHUNK // 16)]

    pltpu.sync_copy(
        tokens_hbm.at[pl.ds(pl.multiple_of(cbase * CHUNK, CHUNK), ROWS_PER_W)],
        idx_all,
    )

    def gather(j, s):
        pltpu.async_copy(
            scr_hbm.at[idx_all.at[pl.ds(j * CHUNK, CHUNK)]], rows[s], gsems[s]
        )

    def wait_gather(s):
        pltpu.make_async_copy(
            scr_hbm.at[pl.ds(0, CHUNK)], rows[s], gsems[s]
        ).wait()

    perm = [(base16 + d) & 15 for d in range(16)]
    fvecs = [base16 + f0 for f0 in (0, 16, 32, 48)]
    zero16 = base16 * 0

    def transpose_chunk(s, st):
        @plsc.parallel_loop(0, CHUNK, step=16, unroll=4)
        def blk(j0):
            jv = base16 + j0
            for i4 in range(4):
                for d in range(16):
                    fv = perm[d] + i4 * 16
                    vals = plsc.load_gather(rows[s], [jv, fv])
                    plsc.store_scatter(tbufs[st], [zero16, fv, jv], vals)

    def put(j, st):
        ch = cbase + j
        l = ch // (BPL * 4)
        b0 = pl.multiple_of((ch % (BPL * 4)) * CHUNK, CHUNK)
        pltpu.async_copy(
            tbufs[st],
            out_hbm.at[pl.ds(l, 1), :, pl.ds(b0, CHUNK)],
            osems[st],
        )

    def wait_put(st):
        pltpu.make_async_copy(
            tbufs[st],
            out_hbm.at[pl.ds(0, 1), :, pl.ds(0, CHUNK)],
            osems[st],
        ).wait()

    gather(0, 0)
    gather(1, 1)

    def body(r, carry):
        for s in range(2):
            j = r * 2 + s
            wait_gather(s)

            @pl.when(j >= 2)
            def _():
                wait_put(s)

            transpose_chunk(s, s)

            @pl.when(j + 2 < NCHUNK)
            def _():
                gather(j + 2, s)

            put(j, s)
        return carry

    lax.fori_loop(0, NCHUNK // 2, body, 0)
    wait_put(0)
    wait_put(1)


def kernel(tokens, table):
    tt = jnp.swapaxes(table, 0, 1)
    tail = table[FULLBLK * 128:].reshape(TAILW * EMB)
    scratch = _transpose_sc(tt, tail)
    flat = jnp.swapaxes(tokens, 0, 1).reshape(NTOK).astype(jnp.int32)
    out = _gather_sc(flat, scratch)
    return jnp.transpose(out, (2, 0, 1))
